# Initial kernel scaffold; baseline (speedup 1.0000x reference)
import jax
import jax.numpy as jnp
from jax.experimental import pallas as pl


def kernel(boxes, cls_preds):
    raise NotImplementedError("write your pallas kernel here")


# speedup vs baseline: 16.7596x; 16.7596x over previous
import functools
import jax
import jax.numpy as jnp
from jax.experimental import pallas as pl
from jax.experimental.pallas import tpu as pltpu

N_BOXES = 20000
NUM_CLASS = 3
N_PRE = 2048
N_POST = 500
THRESH = 0.7
ROWS = N_PRE // 128
OUT_ROWS = 512


def _nms_scan_body(data_ref, out_ref):
    cx = data_ref[0 * ROWS:1 * ROWS, :]
    cy = data_ref[1 * ROWS:2 * ROWS, :]
    w = data_ref[2 * ROWS:3 * ROWS, :]
    h = data_ref[3 * ROWS:4 * ROWS, :]
    sc = data_ref[4 * ROWS:5 * ROWS, :]
    lb = data_ref[5 * ROWS:6 * ROWS, :]

    x1 = cx - 0.5 * w
    y1 = cy - 0.5 * h
    x2 = cx + 0.5 * w
    y2 = cy + 0.5 * h
    area = (x2 - x1) * (y2 - y1)

    row_i = jax.lax.broadcasted_iota(jnp.int32, (ROWS, 128), 0)
    col_i = jax.lax.broadcasted_iota(jnp.int32, (ROWS, 128), 1)
    iota2 = row_i * 128 + col_i
    out_col = jax.lax.broadcasted_iota(jnp.int32, (1, 128), 1)

    def body(i, sup):
        cand = jnp.where(sup != 0, 2 * N_PRE, iota2)
        idx = jnp.min(cand)
        valid = idx < N_PRE
        sel = (iota2 == idx).astype(jnp.float32)

        def ext(p):
            return jnp.sum(p * sel)

        x1s = ext(x1)
        y1s = ext(y1)
        x2s = ext(x2)
        y2s = ext(y2)
        area_s = (x2s - x1s) * (y2s - y1s)

        iw = jnp.clip(jnp.minimum(x2, x2s) - jnp.maximum(x1, x1s), 0.0, None)
        ih = jnp.clip(jnp.minimum(y2, y2s) - jnp.maximum(y1, y1s), 0.0, None)
        inter = iw * ih
        iou = inter / (area + area_s - inter + 1e-8)
        sup = jnp.where(valid, sup | (iou >= THRESH).astype(jnp.int32), sup)

        vf = jnp.where(valid, 1.0, 0.0)
        cxs = ext(cx) * vf
        cys = ext(cy) * vf
        ws = ext(w) * vf
        hs = ext(h) * vf
        scs = ext(sc) * vf
        lbs = (ext(lb) + 1.0) * vf
        row = (jnp.where(out_col == 0, cxs, 0.0)
               + jnp.where(out_col == 1, cys, 0.0)
               + jnp.where(out_col == 2, ws, 0.0)
               + jnp.where(out_col == 3, hs, 0.0)
               + jnp.where(out_col == 4, scs, 0.0)
               + jnp.where(out_col == 5, lbs, 0.0))
        out_ref[pl.ds(i, 1), :] = row
        return sup

    sup0 = jnp.zeros((ROWS, 128), dtype=jnp.int32)
    jax.lax.fori_loop(0, N_POST, body, sup0)


def kernel(boxes, cls_preds):
    scores = jnp.max(cls_preds, axis=1)
    labels = jnp.argmax(cls_preds, axis=1)
    top_scores, top_idx = jax.lax.top_k(scores, N_PRE)
    b = boxes[top_idx]
    lbl = labels[top_idx].astype(jnp.float32)

    planes = jnp.concatenate(
        [
            b[:, 0].reshape(ROWS, 128),
            b[:, 1].reshape(ROWS, 128),
            b[:, 2].reshape(ROWS, 128),
            b[:, 3].reshape(ROWS, 128),
            top_scores.reshape(ROWS, 128),
            lbl.reshape(ROWS, 128),
        ],
        axis=0,
    )

    out = pl.pallas_call(
        _nms_scan_body,
        out_shape=jax.ShapeDtypeStruct((OUT_ROWS, 128), jnp.float32),
    )(planes)

    rois = out[:N_POST, 0:4]
    roi_scores = out[:N_POST, 4]
    roi_labels = out[:N_POST, 5].astype(jnp.int32)
    return rois, roi_scores, roi_labels


# speedup vs baseline: 17.2009x; 1.0263x over previous
import functools
import jax
import jax.numpy as jnp
from jax.experimental import pallas as pl
from jax.experimental.pallas import tpu as pltpu

N_BOXES = 20000
NUM_CLASS = 3
N_PRE = 2048
N_POST = 500
THRESH = 0.7
ROWS = N_PRE // 128
OUT_ROWS = 512


def _nms_scan_body(planes_ref, packed_ref, out_ref):
    cx = planes_ref[0 * ROWS:1 * ROWS, :]
    cy = planes_ref[1 * ROWS:2 * ROWS, :]
    w = planes_ref[2 * ROWS:3 * ROWS, :]
    h = planes_ref[3 * ROWS:4 * ROWS, :]

    x1 = cx - 0.5 * w
    y1 = cy - 0.5 * h
    x2 = cx + 0.5 * w
    y2 = cy + 0.5 * h
    area = (x2 - x1) * (y2 - y1)

    row_i = jax.lax.broadcasted_iota(jnp.int32, (ROWS, 128), 0)
    col_i = jax.lax.broadcasted_iota(jnp.int32, (ROWS, 128), 1)
    iota2 = row_i * 128 + col_i
    lane8 = jax.lax.broadcasted_iota(jnp.int32, (1, 8), 1)

    def body(i, sup):
        cand = jnp.where(sup != 0, 2 * N_PRE, iota2)
        idx = jnp.min(cand)
        valid = idx < N_PRE
        idxc = jnp.minimum(idx, N_PRE - 1)

        d = packed_ref[pl.ds(idxc, 1), :]
        cxs = d[0, 0]
        cys = d[0, 1]
        ws = d[0, 2]
        hs = d[0, 3]
        x1s = cxs - 0.5 * ws
        y1s = cys - 0.5 * hs
        x2s = cxs + 0.5 * ws
        y2s = cys + 0.5 * hs
        area_s = (x2s - x1s) * (y2s - y1s)

        iw = jnp.clip(jnp.minimum(x2, x2s) - jnp.maximum(x1, x1s), 0.0, None)
        ih = jnp.clip(jnp.minimum(y2, y2s) - jnp.maximum(y1, y1s), 0.0, None)
        inter = iw * ih
        iou = inter / (area + area_s - inter + 1e-8)
        sup = jnp.where(valid, sup | (iou >= THRESH).astype(jnp.int32), sup)

        vf = jnp.where(valid, 1.0, 0.0)
        row = (d + jnp.where(lane8 == 5, 1.0, 0.0)) * vf
        out_ref[pl.ds(i, 1), :] = row
        return sup

    sup0 = jnp.zeros((ROWS, 128), dtype=jnp.int32)
    jax.lax.fori_loop(0, N_POST, body, sup0)


def kernel(boxes, cls_preds):
    scores = jnp.max(cls_preds, axis=1)
    labels = jnp.argmax(cls_preds, axis=1)
    top_scores, top_idx = jax.lax.top_k(scores, N_PRE)
    b = boxes[top_idx]
    lbl = labels[top_idx].astype(jnp.float32)

    planes = jnp.concatenate(
        [
            b[:, 0].reshape(ROWS, 128),
            b[:, 1].reshape(ROWS, 128),
            b[:, 2].reshape(ROWS, 128),
            b[:, 3].reshape(ROWS, 128),
        ],
        axis=0,
    )
    packed = jnp.concatenate(
        [b, top_scores[:, None], lbl[:, None],
         jnp.zeros((N_PRE, 2), jnp.float32)],
        axis=1,
    )

    out = pl.pallas_call(
        _nms_scan_body,
        out_shape=jax.ShapeDtypeStruct((OUT_ROWS, 8), jnp.float32),
    )(planes, packed)

    rois = out[:N_POST, 0:4]
    roi_scores = out[:N_POST, 4]
    roi_labels = out[:N_POST, 5].astype(jnp.int32)
    return rois, roi_scores, roi_labels


# speedup vs baseline: 23.2192x; 1.3499x over previous
import functools
import jax
import jax.numpy as jnp
from jax.experimental import pallas as pl
from jax.experimental.pallas import tpu as pltpu

N_BOXES = 20000
NUM_CLASS = 3
N_PRE = 2048
N_POST = 500
THRESH = 0.7
ROWS = N_PRE // 128
OUT_ROWS = 512


def _tree(x, op, axis, size):
    s = 1
    while s < size:
        x = op(x, pltpu.roll(x, s, axis))
        s *= 2
    return x


def _nms_scan_body(planes_ref, out_ref):
    cx = planes_ref[0 * ROWS:1 * ROWS, :]
    cy = planes_ref[1 * ROWS:2 * ROWS, :]
    w = planes_ref[2 * ROWS:3 * ROWS, :]
    h = planes_ref[3 * ROWS:4 * ROWS, :]
    sc = planes_ref[4 * ROWS:5 * ROWS, :]
    lb = planes_ref[5 * ROWS:6 * ROWS, :]

    x1 = cx - 0.5 * w
    y1 = cy - 0.5 * h
    x2 = cx + 0.5 * w
    y2 = cy + 0.5 * h
    area = (x2 - x1) * (y2 - y1)

    row_i = jax.lax.broadcasted_iota(jnp.int32, (ROWS, 128), 0)
    col_i = jax.lax.broadcasted_iota(jnp.int32, (ROWS, 128), 1)
    iota2 = (row_i * 128 + col_i).astype(jnp.float32)
    lane128 = jax.lax.broadcasted_iota(jnp.int32, (1, 128), 1)

    def body(i, sup):
        cand = jnp.where(sup != 0, 2.0 * N_PRE, iota2)
        m = jnp.broadcast_to(jnp.min(cand, axis=1, keepdims=True), (ROWS, 128))
        minb = _tree(m, jnp.minimum, 0, ROWS)
        validv = minb < float(N_PRE)
        sel = ((cand == minb) & validv).astype(jnp.float32)

        def colmax(p):
            return _tree(p * sel, jnp.maximum, 0, ROWS)

        comb = jnp.concatenate(
            [colmax(cx)[0:1], colmax(cy)[1:2], colmax(w)[2:3],
             colmax(h)[3:4], colmax(sc)[4:5], colmax(lb)[5:6],
             jnp.zeros((2, 128), jnp.float32)],
            axis=0,
        )
        combb = jnp.broadcast_to(
            jnp.max(comb, axis=1, keepdims=True), (8, 128))

        cxb = jnp.broadcast_to(combb[0:1, :], (ROWS, 128))
        cyb = jnp.broadcast_to(combb[1:2, :], (ROWS, 128))
        wb = jnp.broadcast_to(combb[2:3, :], (ROWS, 128))
        hb = jnp.broadcast_to(combb[3:4, :], (ROWS, 128))
        x1b = cxb - 0.5 * wb
        y1b = cyb - 0.5 * hb
        x2b = cxb + 0.5 * wb
        y2b = cyb + 0.5 * hb
        area_b = (x2b - x1b) * (y2b - y1b)

        iw = jnp.clip(jnp.minimum(x2, x2b) - jnp.maximum(x1, x1b), 0.0, None)
        ih = jnp.clip(jnp.minimum(y2, y2b) - jnp.maximum(y1, y1b), 0.0, None)
        inter = iw * ih
        iou = inter / (area + area_b - inter + 1e-8)
        supn = sup | ((iou >= THRESH) & validv).astype(jnp.int32)

        vf = jnp.where(validv[0:1, :], 1.0, 0.0)
        shifted = (combb[0:1, :] * (lane128 == 0)
                   + combb[1:2, :] * (lane128 == 1)
                   + combb[2:3, :] * (lane128 == 2)
                   + combb[3:4, :] * (lane128 == 3)
                   + combb[4:5, :] * (lane128 == 4)
                   + (combb[5:6, :] + 1.0) * (lane128 == 5))
        out_ref[pl.ds(i, 1), :] = (shifted * vf)[:, 0:8]
        return supn

    sup0 = jnp.zeros((ROWS, 128), dtype=jnp.int32)
    jax.lax.fori_loop(0, N_POST, body, sup0)


def kernel(boxes, cls_preds):
    scores = jnp.max(cls_preds, axis=1)
    labels = jnp.argmax(cls_preds, axis=1)
    top_scores, top_idx = jax.lax.top_k(scores, N_PRE)
    b = boxes[top_idx]
    lbl = labels[top_idx].astype(jnp.float32)

    planes = jnp.concatenate(
        [
            b[:, 0].reshape(ROWS, 128),
            b[:, 1].reshape(ROWS, 128),
            b[:, 2].reshape(ROWS, 128),
            b[:, 3].reshape(ROWS, 128),
            top_scores.reshape(ROWS, 128),
            lbl.reshape(ROWS, 128),
        ],
        axis=0,
    )

    out = pl.pallas_call(
        _nms_scan_body,
        out_shape=jax.ShapeDtypeStruct((OUT_ROWS, 8), jnp.float32),
    )(planes)

    rois = out[:N_POST, 0:4]
    roi_scores = out[:N_POST, 4]
    roi_labels = out[:N_POST, 5].astype(jnp.int32)
    return rois, roi_scores, roi_labels
